# baseline (speedup 1.0000x reference)
import jax
import jax.numpy as jnp
from jax import lax
from jax.experimental import pallas as pl
from jax.experimental.pallas import tpu as pltpu
from jax.experimental.pallas import tpu_sc as plsc

NB = 8
NA = 8192
NI = 16384
EC = 256
OD = 48
GH = 64
GW = 64
NREQ = GH * GW
MROWS = NB * NA
MPAD = 65 * 1024
ZROW = MROWS
QB = NREQ // 4


def _index_body(idx_hbm, w_hbm, off_hbm, idx1_hbm, wg_hbm, wimg_hbm,
                idxrow, wrow, offv, pos, basebuf, wgbuf, wimgbuf):
    wid = lax.axis_index("s") * 2 + lax.axis_index("c")
    n = wid // 4
    q = wid % 4
    pltpu.sync_copy(idx_hbm.at[pl.ds(n * NA, NA)], idxrow)
    pltpu.sync_copy(w_hbm.at[pl.ds(n * NA, NA)], wrow)
    pltpu.sync_copy(off_hbm, offv)
    nvec = jnp.full((16,), n, jnp.int32)
    h0 = plsc.load_gather(offv, [nvec])
    w0 = plsc.load_gather(offv, [nvec + 16])
    lanes = lax.broadcasted_iota(jnp.int32, (16,), 0)
    qlo = q * QB

    def init_body(k, c):
        pos[pl.ds(k * 16, 16)] = jnp.full((16,), -1, jnp.int32)
        return c
    lax.fori_loop(0, QB // 16, init_body, 0)

    def scat_body(k, c):
        t = idxrow[pl.ds(k * 16, 16)]
        valid = (t >= 0) & (t < NI)
        ti = jnp.right_shift(t, 7)
        tj = t & 127
        ip = ti - h0
        jp = tj - w0
        inw = valid & (ip >= 0) & (ip < GH) & (jp >= 0) & (jp < GW)
        b = ip * GW + jp
        mine = inw & (b >= qlo) & (b < qlo + QB)
        bl = jnp.where(mine, b - qlo, 0)
        plsc.store_scatter(pos, [bl], lanes + k * 16, mask=mine)
        return c
    lax.fori_loop(0, NA // 16, scat_body, 0)

    def req_body(k, c):
        a = pos[pl.ds(k * 16, 16)]
        filled = a >= 0
        ac = jnp.where(filled, a, 0)
        basebuf[pl.ds(k * 16, 16)] = jnp.where(filled, ac + n * NA, ZROW)
        wgbuf[pl.ds(k * 16, 16)] = (plsc.load_gather(wrow, [ac])
                                    * filled.astype(jnp.float32))
        return c
    lax.fori_loop(0, QB // 16, req_body, 0)

    def build_body(il, c):
        for jc in range(16):
            iv = il * 64 + jc * 4 + jnp.right_shift(lanes, 2)
            v = plsc.load_gather(wgbuf, [iv])
            for r in range(4):
                wimgbuf[pl.ds((il * 4 + r) * 256 + jc * 16, 16)] = v
        return c
    lax.fori_loop(0, 16, build_body, 0)

    pltpu.sync_copy(basebuf, idx1_hbm.at[pl.ds(wid * QB, QB)])
    pltpu.sync_copy(wgbuf, wg_hbm.at[pl.ds(wid * QB, QB)])
    pltpu.sync_copy(wimgbuf, wimg_hbm.at[pl.ds(wid * 16384, 16384)])


def _proj_body(emb_ref, wt_ref, b_ref, o_ref):
    s = pl.program_id(0)

    @pl.when(s < 64)
    def _compute():
        acc = jnp.dot(emb_ref[...], wt_ref[...],
                      preferred_element_type=jnp.float32)
        o_ref[...] = acc + b_ref[...]

    @pl.when(s == 64)
    def _pad():
        o_ref[...] = jnp.zeros_like(o_ref)


def _copy_body(i_ref, o_ref):
    o_ref[...] = i_ref[...]


def _gather_body(xs_hbm, idx1_hbm, wg_hbm, xg_hbm, idxv, wgv, rows, staging,
                 sem):
    wid = lax.axis_index("s") * 2 + lax.axis_index("c")
    pltpu.sync_copy(idx1_hbm.at[pl.ds(wid * 8, 8)], idxv)
    pltpu.sync_copy(wg_hbm.at[pl.ds(wid * QB, QB)], wgv)
    copies = [pltpu.async_copy(xs_hbm.at[idxv.at[jj]],
                               rows.at[pl.ds(jj * 128, 128)], sem)
              for jj in range(8)]
    for cp in copies:
        cp.wait()

    def scale_body(j, c):
        w = plsc.load_gather(wgv, [jnp.full((16,), j, jnp.int32)])
        for v in range(3):
            staging[pl.ds(j * OD + v * 16, 16)] = rows[j, pl.ds(v * 16, 16)] * w
        return c
    lax.fori_loop(0, QB, scale_body, 0)

    pltpu.sync_copy(staging, xg_hbm.at[pl.ds(wid * (QB * OD), QB * OD)])


def kernel(emb, indexes, weights, img_size, h_offset, w_offset, W, b):
    size_zero = (jnp.asarray(img_size[0], jnp.int32)
                 + jnp.asarray(img_size[1], jnp.int32) - jnp.int32(512))
    h0p = (h_offset.astype(jnp.int32) + size_zero) // 4
    w0p = (w_offset.astype(jnp.int32) + size_zero) // 4
    zpad = jnp.zeros((8,), jnp.int32)
    offs = jnp.concatenate([h0p, zpad, w0p, zpad,
                            jnp.zeros((96,), jnp.int32)])

    mesh = plsc.VectorSubcoreMesh(core_axis_name="c", subcore_axis_name="s")
    sc_params = pltpu.CompilerParams(needs_layout_passes=False,
                                     use_tc_tiling_on_sc=False)
    idx_kernel = pl.kernel(
        _index_body,
        mesh=mesh,
        compiler_params=sc_params,
        out_type=[
            jax.ShapeDtypeStruct((32 * QB,), jnp.int32),
            jax.ShapeDtypeStruct((32 * QB,), jnp.float32),
            jax.ShapeDtypeStruct((32 * 16384,), jnp.float32),
        ],
        scratch_types=[
            pltpu.VMEM((NA,), jnp.int32),
            pltpu.VMEM((NA,), jnp.float32),
            pltpu.VMEM((128,), jnp.int32),
            pltpu.VMEM((QB,), jnp.int32),
            pltpu.VMEM((QB,), jnp.int32),
            pltpu.VMEM((QB,), jnp.float32),
            pltpu.VMEM((16384,), jnp.float32),
        ],
    )
    idx1, wg, wimg = idx_kernel(indexes.reshape(MROWS), weights.reshape(MROWS),
                                offs)

    emb2 = emb.reshape(MROWS, EC)
    xs = pl.pallas_call(
        _proj_body,
        grid=(65,),
        in_specs=[
            pl.BlockSpec((1024, EC), lambda s: (jnp.minimum(s, 63), 0)),
            pl.BlockSpec((EC, OD), lambda s: (0, 0)),
            pl.BlockSpec((1, OD), lambda s: (0, 0)),
        ],
        out_specs=pl.BlockSpec((1024, OD), lambda s: (s, 0)),
        out_shape=jax.ShapeDtypeStruct((MPAD, OD), jnp.float32),
    )(emb2, W.T, b.reshape(1, OD))

    idx1r = idx1.reshape(256, 128)
    gather_kernel = pl.kernel(
        _gather_body,
        mesh=mesh,
        compiler_params=sc_params,
        out_type=jax.ShapeDtypeStruct((NB * NREQ * OD,), jnp.float32),
        scratch_types=[
            pltpu.VMEM((8, 128), jnp.int32),
            pltpu.VMEM((QB,), jnp.float32),
            pltpu.VMEM((QB, OD), jnp.float32),
            pltpu.VMEM((QB * OD,), jnp.float32),
            pltpu.SemaphoreType.DMA,
        ],
    )
    xg = gather_kernel(xs, idx1r, wg)
    xg = pl.pallas_call(
        _copy_body,
        grid=(NB,),
        in_specs=[pl.BlockSpec((1536, 128), lambda n: (n, 0))],
        out_specs=pl.BlockSpec((1536, 128), lambda n: (n, 0)),
        out_shape=jax.ShapeDtypeStruct((NB * 1536, 128), jnp.float32),
    )(xg.reshape(NB * 1536, 128))
    x6 = xg.reshape(NB, GH, GW, 4, 4, 3)
    img = jnp.transpose(x6, (0, 1, 3, 2, 4, 5)).reshape(NB, 256, 256, 3)
    weight_img = wimg.reshape(NB, 256, 256)
    return img, weight_img


# speedup vs baseline: 2.5686x; 2.5686x over previous
import jax
import jax.numpy as jnp
from jax import lax
from jax.experimental import pallas as pl
from jax.experimental.pallas import tpu as pltpu
from jax.experimental.pallas import tpu_sc as plsc

NB = 8
NA = 8192
NI = 16384
EC = 256
OD = 48
GH = 64
GW = 64
NREQ = GH * GW
MROWS = NB * NA
MPAD = 65 * 1024
ZROW = MROWS
QB = NREQ // 4


def _index_body(idx_hbm, w_hbm, off_hbm, idx1_hbm, wg_hbm, wimg_hbm,
                idxrow, wrow, offv, pos, basebuf, wgbuf, wimgbuf):
    wid = lax.axis_index("s") * 2 + lax.axis_index("c")
    n = wid // 4
    q = wid % 4
    pltpu.sync_copy(idx_hbm.at[pl.ds(n * NA, NA)], idxrow)
    pltpu.sync_copy(w_hbm.at[pl.ds(n * NA, NA)], wrow)
    pltpu.sync_copy(off_hbm, offv)
    nvec = jnp.full((16,), n, jnp.int32)
    h0 = plsc.load_gather(offv, [nvec])
    w0 = plsc.load_gather(offv, [nvec + 16])
    lanes = lax.broadcasted_iota(jnp.int32, (16,), 0)
    qlo = q * QB

    def init_body(k, c):
        pos[pl.ds(k * 16, 16)] = jnp.full((16,), -1, jnp.int32)
        return c
    lax.fori_loop(0, QB // 16, init_body, 0)

    def scat_body(k, c):
        t = idxrow[pl.ds(k * 16, 16)]
        valid = (t >= 0) & (t < NI)
        ti = jnp.right_shift(t, 7)
        tj = t & 127
        ip = ti - h0
        jp = tj - w0
        inw = valid & (ip >= 0) & (ip < GH) & (jp >= 0) & (jp < GW)
        b = ip * GW + jp
        mine = inw & (b >= qlo) & (b < qlo + QB)
        bl = jnp.where(mine, b - qlo, 0)
        plsc.store_scatter(pos, [bl], lanes + k * 16, mask=mine)
        return c
    lax.fori_loop(0, NA // 16, scat_body, 0)

    def req_body(k, c):
        a = pos[pl.ds(k * 16, 16)]
        filled = a >= 0
        ac = jnp.where(filled, a, 0)
        basebuf[pl.ds(k * 16, 16)] = jnp.where(filled, ac + n * NA, ZROW)
        wgbuf[pl.ds(k * 16, 16)] = (plsc.load_gather(wrow, [ac])
                                    * filled.astype(jnp.float32))
        return c
    lax.fori_loop(0, QB // 16, req_body, 0)

    def build_body(il, c):
        for jc in range(16):
            iv = il * 64 + jc * 4 + jnp.right_shift(lanes, 2)
            v = plsc.load_gather(wgbuf, [iv])
            for r in range(4):
                wimgbuf[pl.ds((il * 4 + r) * 256 + jc * 16, 16)] = v
        return c
    lax.fori_loop(0, 16, build_body, 0)

    pltpu.sync_copy(basebuf, idx1_hbm.at[pl.ds(wid * QB, QB)])
    pltpu.sync_copy(wgbuf, wg_hbm.at[pl.ds(wid * QB, QB)])
    pltpu.sync_copy(wimgbuf, wimg_hbm.at[pl.ds(wid * 16384, 16384)])


def _proj_body(emb_ref, wt_ref, b_ref, o_ref):
    s = pl.program_id(0)

    @pl.when(s < 64)
    def _compute():
        acc = jnp.dot(emb_ref[...], wt_ref[...],
                      preferred_element_type=jnp.float32)
        o_ref[...] = acc + b_ref[...]

    @pl.when(s == 64)
    def _pad():
        o_ref[...] = jnp.zeros_like(o_ref)


def _copy_body(i_ref, o_ref):
    o_ref[...] = i_ref[...].reshape(1, 1, 256, 256)


def _gather_body(xs_hbm, idx1_hbm, wg_hbm, xg_hbm, idxv, wgv, rows, staging,
                 sem):
    wid = lax.axis_index("s") * 2 + lax.axis_index("c")
    pltpu.sync_copy(idx1_hbm.at[pl.ds(wid * 8, 8)], idxv)
    pltpu.sync_copy(wg_hbm.at[pl.ds(wid * QB, QB)], wgv)
    copies = [pltpu.async_copy(xs_hbm.at[idxv.at[jj]],
                               rows.at[pl.ds(jj * 128, 128)], sem)
              for jj in range(8)]
    for cp in copies:
        cp.wait()

    lanes = lax.broadcasted_iota(jnp.int32, (16,), 0)
    pixc = (lanes & 3) * 3
    n = wid // 4
    q = wid % 4

    def reorder_body(il, cr):
        for jc in range(16):
            rowv = il * 64 + jc * 4 + jnp.right_shift(lanes, 2)
            wv = plsc.load_gather(wgv, [rowv])
            for r in range(4):
                for c in range(3):
                    vals = plsc.load_gather(rows, [rowv, r * 12 + pixc + c])
                    staging[pl.ds(c * 16384 + (il * 4 + r) * 256 + jc * 16,
                                  16)] = vals * wv
        return cr
    lax.fori_loop(0, 16, reorder_body, 0)

    for c in range(3):
        pltpu.sync_copy(
            staging.at[pl.ds(c * 16384, 16384)],
            xg_hbm.at[pl.ds(n * 196608 + c * 65536 + q * 16384, 16384)])


def kernel(emb, indexes, weights, img_size, h_offset, w_offset, W, b):
    size_zero = (jnp.asarray(img_size[0], jnp.int32)
                 + jnp.asarray(img_size[1], jnp.int32) - jnp.int32(512))
    h0p = (h_offset.astype(jnp.int32) + size_zero) // 4
    w0p = (w_offset.astype(jnp.int32) + size_zero) // 4
    zpad = jnp.zeros((8,), jnp.int32)
    offs = jnp.concatenate([h0p, zpad, w0p, zpad,
                            jnp.zeros((96,), jnp.int32)])

    mesh = plsc.VectorSubcoreMesh(core_axis_name="c", subcore_axis_name="s")
    sc_params = pltpu.CompilerParams(needs_layout_passes=False,
                                     use_tc_tiling_on_sc=False)
    idx_kernel = pl.kernel(
        _index_body,
        mesh=mesh,
        compiler_params=sc_params,
        out_type=[
            jax.ShapeDtypeStruct((32 * QB,), jnp.int32),
            jax.ShapeDtypeStruct((32 * QB,), jnp.float32),
            jax.ShapeDtypeStruct((32 * 16384,), jnp.float32),
        ],
        scratch_types=[
            pltpu.VMEM((NA,), jnp.int32),
            pltpu.VMEM((NA,), jnp.float32),
            pltpu.VMEM((128,), jnp.int32),
            pltpu.VMEM((QB,), jnp.int32),
            pltpu.VMEM((QB,), jnp.int32),
            pltpu.VMEM((QB,), jnp.float32),
            pltpu.VMEM((16384,), jnp.float32),
        ],
    )
    idx1, wg, wimg = idx_kernel(indexes.reshape(MROWS), weights.reshape(MROWS),
                                offs)

    emb2 = emb.reshape(MROWS, EC)
    xs = pl.pallas_call(
        _proj_body,
        grid=(65,),
        in_specs=[
            pl.BlockSpec((1024, EC), lambda s: (jnp.minimum(s, 63), 0)),
            pl.BlockSpec((EC, OD), lambda s: (0, 0)),
            pl.BlockSpec((1, OD), lambda s: (0, 0)),
        ],
        out_specs=pl.BlockSpec((1024, OD), lambda s: (s, 0)),
        out_shape=jax.ShapeDtypeStruct((MPAD, OD), jnp.float32),
    )(emb2, W.T, b.reshape(1, OD))

    idx1r = idx1.reshape(256, 128)
    gather_kernel = pl.kernel(
        _gather_body,
        mesh=mesh,
        compiler_params=sc_params,
        out_type=jax.ShapeDtypeStruct((NB * NREQ * OD,), jnp.float32),
        scratch_types=[
            pltpu.VMEM((8, 128), jnp.int32),
            pltpu.VMEM((QB,), jnp.float32),
            pltpu.VMEM((QB, OD), jnp.float32),
            pltpu.VMEM((QB * OD,), jnp.float32),
            pltpu.SemaphoreType.DMA,
        ],
    )
    imgp = gather_kernel(xs, idx1r, wg)
    planes = pl.pallas_call(
        _copy_body,
        grid=(NB, 3),
        in_specs=[pl.BlockSpec((512, 128), lambda n, c: (n * 3 + c, 0))],
        out_specs=pl.BlockSpec((1, 1, 256, 256), lambda n, c: (n, c, 0, 0)),
        out_shape=jax.ShapeDtypeStruct((NB, 3, 256, 256), jnp.float32),
    )(imgp.reshape(NB * 3 * 512, 128))
    img = jnp.transpose(planes, (0, 2, 3, 1))
    weight_img = wimg.reshape(NB, 256, 256)
    return img, weight_img
